# baseline (device time: 12080 ns/iter reference)
import jax
import jax.numpy as jnp
from jax import lax
from jax.experimental import pallas as pl
from jax.experimental.pallas import tpu as pltpu

K = 8


def kernel(x):
    m, n = x.shape
    ch = m // K
    half = K // 2

    def body(x_hbm, out_ref, buf, acc, comm_ref, dma_sems, send_sems, recv_sems):
        my_x = lax.axis_index("x")
        my_y = lax.axis_index("y")
        nbr = (1 - my_x, my_y)

        barrier_sem = pltpu.get_barrier_semaphore()
        pl.semaphore_signal(
            barrier_sem, inc=1, device_id=nbr,
            device_id_type=pl.DeviceIdType.MESH,
        )

        def chunk_copy(k):
            return pltpu.make_async_copy(
                x_hbm.at[pl.ds(k * ch, ch), :], buf.at[k], dma_sems.at[k]
            )

        for k in range(K):
            chunk_copy(k).start()

        def exchange(slot):
            return pltpu.make_async_remote_copy(
                src_ref=comm_ref.at[2 * slot],
                dst_ref=comm_ref.at[2 * slot + 1],
                send_sem=send_sems.at[slot],
                recv_sem=recv_sems.at[slot],
                device_id=nbr,
                device_id_type=pl.DeviceIdType.MESH,
            )

        chunk_copy(0).wait()
        acc[...] = jnp.sum(buf[0], axis=0, keepdims=True)
        for k in range(1, half):
            chunk_copy(k).wait()
            acc[...] += jnp.sum(buf[k], axis=0, keepdims=True)

        comm_ref[0] = acc[...]
        pl.semaphore_wait(barrier_sem, 1)
        exchange(0).start()

        chunk_copy(half).wait()
        acc[...] = jnp.sum(buf[half], axis=0, keepdims=True)
        for k in range(half + 1, K):
            chunk_copy(k).wait()
            acc[...] += jnp.sum(buf[k], axis=0, keepdims=True)

        comm_ref[2] = acc[...]
        rdma2 = exchange(1)
        rdma2.start()
        rdma1 = exchange(0)
        rdma1.wait()
        partial = comm_ref[0] + comm_ref[1] + comm_ref[2]
        rdma2.wait()
        out_ref[...] = partial + comm_ref[3]

    return pl.pallas_call(
        body,
        out_shape=jax.ShapeDtypeStruct((1, n), jnp.float32),
        in_specs=[pl.BlockSpec(memory_space=pl.ANY)],
        out_specs=pl.BlockSpec(memory_space=pltpu.VMEM),
        scratch_shapes=[
            pltpu.VMEM((K, ch, n), jnp.float32),
            pltpu.VMEM((1, n), jnp.float32),
            pltpu.VMEM((4, 1, n), jnp.float32),
            pltpu.SemaphoreType.DMA((K,)),
            pltpu.SemaphoreType.DMA((2,)),
            pltpu.SemaphoreType.DMA((2,)),
        ],
        compiler_params=pltpu.CompilerParams(collective_id=0),
    )(x)


# device time: 10739 ns/iter; 1.1249x vs baseline; 1.1249x over previous
import jax
import jax.numpy as jnp
from jax import lax
from jax.experimental import pallas as pl
from jax.experimental.pallas import tpu as pltpu


def kernel(x):
    m, n = x.shape
    half = m // 2

    def body(x_ref, out_ref, comm_ref, send_sems, recv_sems):
        my_x = lax.axis_index("x")
        my_y = lax.axis_index("y")
        nbr = (1 - my_x, my_y)

        barrier_sem = pltpu.get_barrier_semaphore()
        pl.semaphore_signal(
            barrier_sem, inc=1, device_id=nbr,
            device_id_type=pl.DeviceIdType.MESH,
        )

        def exchange(slot):
            return pltpu.make_async_remote_copy(
                src_ref=comm_ref.at[2 * slot],
                dst_ref=comm_ref.at[2 * slot + 1],
                send_sem=send_sems.at[slot],
                recv_sem=recv_sems.at[slot],
                device_id=nbr,
                device_id_type=pl.DeviceIdType.MESH,
            )

        comm_ref[0] = jnp.sum(x_ref[:half, :], axis=0, keepdims=True)
        pl.semaphore_wait(barrier_sem, 1)
        rdma1 = exchange(0)
        rdma1.start()

        comm_ref[2] = jnp.sum(x_ref[half:, :], axis=0, keepdims=True)
        rdma2 = exchange(1)
        rdma2.start()

        rdma1.wait()
        partial = comm_ref[0] + comm_ref[1] + comm_ref[2]
        rdma2.wait()
        out_ref[...] = partial + comm_ref[3]

    return pl.pallas_call(
        body,
        out_shape=jax.ShapeDtypeStruct((1, n), jnp.float32),
        in_specs=[pl.BlockSpec(memory_space=pltpu.VMEM)],
        out_specs=pl.BlockSpec(memory_space=pltpu.VMEM),
        scratch_shapes=[
            pltpu.VMEM((4, 1, n), jnp.float32),
            pltpu.SemaphoreType.DMA((2,)),
            pltpu.SemaphoreType.DMA((2,)),
        ],
        compiler_params=pltpu.CompilerParams(collective_id=0),
    )(x)
